# baseline (speedup 1.0000x reference)
import functools

import jax
import jax.numpy as jnp
from jax import lax
from jax.experimental import pallas as pl
from jax.experimental.pallas import tpu as pltpu
from jax.experimental.pallas import tpu_sc as plsc

_D = 64
_SCALE = 8.0

_NC = 2
_NS = 16
_NW = _NC * _NS

_CTOK = 16
_NBUF = 2


@functools.lru_cache(maxsize=None)
def _emb_fn(n_tok, seq):
    tok_per_w = n_tok // _NW
    rows_per_chunk = _CTOK * seq
    n_chunks = tok_per_w // _CTOK
    n_groups = n_chunks // _NBUF
    mesh = plsc.VectorSubcoreMesh(core_axis_name="c", subcore_axis_name="s")

    scratch = [
        pltpu.VMEM((tok_per_w, seq), jnp.int32),
        pltpu.VMEM((n_chunks, rows_per_chunk), jnp.int32),
    ]
    scratch += [
        pltpu.VMEM((rows_per_chunk, _D), jnp.float32) for _ in range(_NBUF)
    ]
    scratch += [
        pltpu.VMEM((_CTOK, seq, _D), jnp.float32) for _ in range(_NBUF)
    ]
    scratch += [pltpu.SemaphoreType.DMA for _ in range(2 * _NBUF + 1)]

    @functools.partial(
        pl.kernel,
        mesh=mesh,
        compiler_params=pltpu.CompilerParams(use_tc_tiling_on_sc=False),
        out_type=jax.ShapeDtypeStruct((n_tok, seq, _D), jnp.float32),
        scratch_types=scratch,
    )
    def emb(table_hbm, x_hbm, out_hbm, idx2d, idxf, *rest):
        gbufs = rest[:_NBUF]
        sbufs = rest[_NBUF:2 * _NBUF]
        gsem = rest[2 * _NBUF:3 * _NBUF]
        osem = rest[3 * _NBUF:4 * _NBUF]
        xsem = rest[4 * _NBUF]

        wid = lax.axis_index("s") * _NC + lax.axis_index("c")
        tok0 = wid * tok_per_w
        pltpu.async_copy(
            x_hbm.at[pl.ds(tok0, tok_per_w), :], idx2d, xsem
        ).wait()

        def flat_body(c, carry):
            for t in range(_CTOK):
                row = c * _CTOK + t
                a = idx2d[row, pl.ds(0, 16)]
                bvec = idx2d[row, pl.ds(seq - 16, 16)]
                idxf[c, pl.ds(t * seq, 16)] = a
                idxf[c, pl.ds(t * seq + seq - 16, 16)] = bvec
            return carry

        lax.fori_loop(0, n_chunks, flat_body, 0)

        def scale_buf(gbuf, sbuf):
            def body(i, carry):
                for j in range(seq):
                    for k in range(_D // 16):
                        sl = pl.ds(k * 16, 16)
                        sbuf[i, j, sl] = gbuf[i * seq + j, sl] * _SCALE
                return carry

            lax.fori_loop(0, _CTOK, body, 0)

        def group_body(g, carry):
            c0 = g * _NBUF
            for b in range(_NBUF):
                t_off = (c0 + b) * _CTOK
                dst = out_hbm.at[pl.ds(tok0 + t_off, _CTOK)]

                @pl.when(g != 0)
                def _drain():
                    pltpu.make_async_copy(sbufs[b], dst, osem[b]).wait()

                pltpu.async_copy(
                    table_hbm.at[idxf.at[c0 + b]],
                    gbufs[b],
                    gsem[b],
                )
            for b in range(_NBUF):
                t_off = (c0 + b) * _CTOK
                dst = out_hbm.at[pl.ds(tok0 + t_off, _CTOK)]
                pltpu.make_async_copy(
                    table_hbm.at[idxf.at[c0 + b]],
                    gbufs[b],
                    gsem[b],
                ).wait()
                scale_buf(gbufs[b], sbufs[b])
                pltpu.async_copy(sbufs[b], dst, osem[b])
            return carry

        lax.fori_loop(0, n_groups, group_body, 0)
        for b in range(_NBUF):
            t_off = ((n_groups - 1) * _NBUF + b) * _CTOK
            dst = out_hbm.at[pl.ds(tok0 + t_off, _CTOK)]
            pltpu.make_async_copy(sbufs[b], dst, osem[b]).wait()

    return emb


def kernel(x, embedding_weight):
    n_tok, seq = x.shape
    return _emb_fn(n_tok, seq)(embedding_weight, x)


# speedup vs baseline: 1.0497x; 1.0497x over previous
import functools

import jax
import jax.numpy as jnp
from jax import lax
from jax.experimental import pallas as pl
from jax.experimental.pallas import tpu as pltpu
from jax.experimental.pallas import tpu_sc as plsc

_D = 64
_SCALE = 8.0

_NC = 2
_NS = 16
_NW = _NC * _NS

_NBUF = 2
_UNROLL = 4


@functools.lru_cache(maxsize=None)
def _emb_fn(n_tok, seq):
    tok_per_w = n_tok // _NW
    mesh = plsc.VectorSubcoreMesh(core_axis_name="c", subcore_axis_name="s")

    scratch = [pltpu.VMEM((seq, tok_per_w), jnp.int32)]
    scratch += [
        pltpu.VMEM((tok_per_w, _D), jnp.float32) for _ in range(_NBUF)
    ]
    scratch += [pltpu.SemaphoreType.DMA for _ in range(2 * _NBUF + 1)]

    @functools.partial(
        pl.kernel,
        mesh=mesh,
        compiler_params=pltpu.CompilerParams(use_tc_tiling_on_sc=False),
        out_type=jax.ShapeDtypeStruct((seq, n_tok, _D), jnp.float32),
        scratch_types=scratch,
    )
    def emb(table_hbm, xt_hbm, out_hbm, xbuf, *rest):
        bufs = rest[:_NBUF]
        gsem = rest[_NBUF:2 * _NBUF]
        osem = rest[2 * _NBUF:3 * _NBUF]
        xsem = rest[3 * _NBUF]

        wid = lax.axis_index("s") * _NC + lax.axis_index("c")
        tok0 = wid * tok_per_w
        pltpu.async_copy(
            xt_hbm.at[:, pl.ds(tok0, tok_per_w)], xbuf, xsem
        ).wait()

        def scale_buf(buf):
            def body(i, carry):
                r0 = i * _UNROLL
                for dr in range(_UNROLL):
                    for k in range(_D // 16):
                        sl = pl.ds(k * 16, 16)
                        buf[r0 + dr, sl] = buf[r0 + dr, sl] * _SCALE
                return carry

            lax.fori_loop(0, tok_per_w // _UNROLL, body, 0)

        def pair_body(g, carry):
            for b in range(_NBUF):
                s = g * _NBUF + b
                dst = out_hbm.at[s, pl.ds(tok0, tok_per_w), :]

                @pl.when(g != 0)
                def _drain():
                    pltpu.make_async_copy(bufs[b], dst, osem[b]).wait()

                pltpu.async_copy(
                    table_hbm.at[xbuf.at[s]], bufs[b], gsem[b]
                )
            for b in range(_NBUF):
                s = g * _NBUF + b
                dst = out_hbm.at[s, pl.ds(tok0, tok_per_w), :]
                pltpu.make_async_copy(
                    table_hbm.at[xbuf.at[s]], bufs[b], gsem[b]
                ).wait()
                scale_buf(bufs[b])
                pltpu.async_copy(bufs[b], dst, osem[b])
            return carry

        lax.fori_loop(0, seq // _NBUF, pair_body, 0)
        for b in range(_NBUF):
            s = seq - _NBUF + b
            dst = out_hbm.at[s, pl.ds(tok0, tok_per_w), :]
            pltpu.make_async_copy(bufs[b], dst, osem[b]).wait()

    return emb


def kernel(x, embedding_weight):
    n_tok, seq = x.shape
    out_p = _emb_fn(n_tok, seq)(embedding_weight, x.T)
    return out_p.transpose(1, 0, 2)
